# baseline (speedup 1.0000x reference)
import functools

import jax
import jax.numpy as jnp
from jax import lax
from jax.experimental import pallas as pl
from jax.experimental.pallas import tpu as pltpu
from jax.experimental.pallas import tpu_sc as plsc

_D = 128
_N = 50000
_E = 500000
_G = 64
_STEPS = 3

_BR = 1000
_GRID = _N // _BR

_EB = 128
_NTILES = 16
_GBLK = 2
_GTRIPS = 42
_GEDGES = _GBLK * _EB
_EPAD = _NTILES * 3 * _GTRIPS * _GEDGES
_ACC_ROWS = 50048
_ZROWS = _ACC_ROWS // _NTILES
_WOUT = 125
_OUT_ROWS = _N + 8


def _seg_body(m2, srcp, dstp, zsrc, out2, acc, sg, dg,
              idxb, dstb, rowsb, gsem, ssem, isem):
    cid = lax.axis_index("c")
    tid = lax.axis_index("s")
    iota = lax.iota(jnp.int32, 16)

    for p in range(2):
        k = cid * 2 + p
        pltpu.sync_copy(zsrc, acc.at[pl.ds(tid * _ZROWS, _ZROWS)])
        plsc.subcore_barrier()

        def stage(g, si):
            base = (g * _NTILES + tid) * _GEDGES
            pltpu.async_copy(srcp.at[pl.ds(base, _GEDGES)], sg[si], isem[si])
            pltpu.async_copy(dstp.at[pl.ds(base, _GEDGES)], dg[si], isem[si])

        def wait_stage(si):
            pltpu.make_async_copy(srcp.at[pl.ds(0, _GEDGES)], sg[si],
                                  isem[si]).wait()
            pltpu.make_async_copy(dstp.at[pl.ds(0, _GEDGES)], dg[si],
                                  isem[si]).wait()

        def drain_scatter(b):
            pltpu.make_async_copy(m2.at[pl.ds(0, _EB)], rowsb[b],
                                  ssem[b]).wait()

        def fire_set(i, si):
            for b in (2 * si, 2 * si + 1):
                @pl.when(i > 0)
                def _():
                    drain_scatter(b)
                for j in range(_EB // 16):
                    o = (b - 2 * si) * _EB + j * 16
                    s = sg[si][pl.ds(o, 16)]
                    idxb[b][pl.ds(j * 16, 16)] = s * 4 + k
                    dstb[b][pl.ds(j * 16, 16)] = dg[si][pl.ds(o, 16)]
                pltpu.async_copy(m2.at[idxb[b]], rowsb[b], gsem[b])

        def add_set(si):
            for b in (2 * si, 2 * si + 1):
                pltpu.make_async_copy(m2.at[pl.ds(0, _EB)], rowsb[b],
                                      gsem[b]).wait()
                pltpu.async_copy(rowsb[b], acc.at[dstb[b]], ssem[b], add=True)

        for si in range(3):
            stage(si, si)

        def edge_triple(i, _):
            for si in range(3):
                wait_stage(si)
                fire_set(i, si)

            @pl.when(i < _GTRIPS - 1)
            def _():
                for si in range(3):
                    stage(3 * i + 3 + si, si)

            for si in range(3):
                add_set(si)
            return 0

        lax.fori_loop(0, _GTRIPS, edge_triple, 0)
        for b in range(6):
            drain_scatter(b)
        plsc.subcore_barrier()

        def wdrain(buf, sem):
            pltpu.make_async_copy(m2.at[pl.ds(0, _EB)], buf, sem).wait()

        def write_block(i, buf, bidx, sem, first):
            row0 = tid * (25 * _WOUT) + i * _WOUT
            if first is None:
                @pl.when(i > 1)
                def _():
                    wdrain(buf, sem)
            elif not first:
                wdrain(buf, sem)
            pltpu.sync_copy(acc.at[pl.ds(row0, _EB)], buf)
            for j in range(_EB // 16):
                r = iota + (j * 16)
                v = (row0 + r) * 4 + k
                bidx[pl.ds(j * 16, 16)] = jnp.where(
                    r < _WOUT, v, _N * 4 + tid)
            pltpu.async_copy(buf, out2.at[bidx], sem)

        def write_pair(i, _):
            write_block(2 * i, rowsb[0], idxb[0], ssem[0], None)
            write_block(2 * i + 1, rowsb[1], idxb[1], ssem[1], None)
            return 0

        lax.fori_loop(0, 12, write_pair, 0)
        write_block(24, rowsb[0], idxb[0], ssem[0], False)
        wdrain(rowsb[1], ssem[1])
        wdrain(rowsb[0], ssem[0])
        plsc.subcore_barrier()


@functools.cache
def _seg_call():
    return functools.partial(
        pl.kernel,
        out_type=jax.ShapeDtypeStruct((_OUT_ROWS * 4, 32), jnp.float32),
        mesh=plsc.VectorSubcoreMesh(core_axis_name="c", subcore_axis_name="s"),
        compiler_params=pltpu.CompilerParams(use_tc_tiling_on_sc=False),
        scratch_types=[
            pltpu.VMEM_SHARED((_ACC_ROWS, 32), jnp.float32),
            [pltpu.VMEM((_GEDGES,), jnp.int32) for _ in range(3)],
            [pltpu.VMEM((_GEDGES,), jnp.int32) for _ in range(3)],
            [pltpu.VMEM((_EB,), jnp.int32) for _ in range(6)],
            [pltpu.VMEM((_EB,), jnp.int32) for _ in range(6)],
            [pltpu.VMEM((_EB, 32), jnp.float32) for _ in range(6)],
            [pltpu.SemaphoreType.DMA for _ in range(6)],
            [pltpu.SemaphoreType.DMA for _ in range(6)],
            [pltpu.SemaphoreType.DMA for _ in range(3)],
        ],
    )(_seg_body)


def _segment_sum(m, srcp, dstp, zsrc):
    m2 = m.reshape(_N * 4, 32)
    out2 = _seg_call()(m2, srcp, dstp, zsrc)
    return out2.reshape(_OUT_ROWS, _D)


def _full(shape):
    return pl.BlockSpec(shape, lambda i: (0, 0))


def _rows(width):
    return pl.BlockSpec((_BR, width), lambda i: (i, 0))


def _embed_body(xf_ref, emb_ref, o_ref):
    xf = xf_ref[...]
    e0 = emb_ref[0:1, :]
    e1 = emb_ref[1:2, :]
    o_ref[...] = e0 + xf * (e1 - e0)


def _embed(xf, emb):
    return pl.pallas_call(
        _embed_body,
        grid=(_GRID,),
        in_specs=[_rows(1), _full((2, _D))],
        out_specs=_rows(_D),
        out_shape=jax.ShapeDtypeStruct((_N, _D), jnp.float32),
    )(xf, emb)


def _bdot(x, w):
    return jnp.dot(x.astype(jnp.bfloat16), w.astype(jnp.bfloat16),
                   preferred_element_type=jnp.float32)


def _elayer_body(x_ref, w1, b1, w2, b2, w3, b3, o_ref):
    x = x_ref[...]
    x = jnp.maximum(_bdot(x, w1[...]) + b1[...], 0.0)
    x = jnp.maximum(_bdot(x, w2[...]) + b2[...], 0.0)
    x = jnp.maximum(_bdot(x, w3[...]) + b3[...], 0.0)
    o_ref[...] = x


def _elayer(ep, x):
    return pl.pallas_call(
        _elayer_body,
        grid=(_GRID,),
        in_specs=[_rows(_D), _full((_D, _D)), _full((1, _D)),
                  _full((_D, _D)), _full((1, _D)),
                  _full((_D, _D)), _full((1, _D))],
        out_specs=_rows(_D),
        out_shape=jax.ShapeDtypeStruct((_N, _D), jnp.float32),
    )(x, ep["W1"], ep["b1"].reshape(1, _D), ep["W2"], ep["b2"].reshape(1, _D),
      ep["W3"], ep["b3"].reshape(1, _D))


def _lstm_body(x_ref, h_ref, c_ref, wih, whh, b, h_ref_o, c_ref_o):
    gates = _bdot(x_ref[...], wih[...]) + _bdot(h_ref[...], whh[...]) + b[...]
    i = jax.nn.sigmoid(gates[:, 0:_D])
    f = jax.nn.sigmoid(gates[:, _D:2 * _D])
    g = jnp.tanh(gates[:, 2 * _D:3 * _D])
    o = jax.nn.sigmoid(gates[:, 3 * _D:4 * _D])
    c2 = f * c_ref[...] + i * g
    h_ref_o[...] = o * jnp.tanh(c2)
    c_ref_o[...] = jnp.maximum(c2, 0.0)


def _lstm_el_body(x_ref, h_ref, c_ref, wih, whh, b, w1, b1, w2, b2, w3, b3,
                  h_ref_o, c_ref_o, m_ref_o):
    gates = _bdot(x_ref[...], wih[...]) + _bdot(h_ref[...], whh[...]) + b[...]
    i = jax.nn.sigmoid(gates[:, 0:_D])
    f = jax.nn.sigmoid(gates[:, _D:2 * _D])
    g = jnp.tanh(gates[:, 2 * _D:3 * _D])
    o = jax.nn.sigmoid(gates[:, 3 * _D:4 * _D])
    c2 = f * c_ref[...] + i * g
    h_ref_o[...] = o * jnp.tanh(c2)
    x = jnp.maximum(c2, 0.0)
    c_ref_o[...] = x
    x = jnp.maximum(_bdot(x, w1[...]) + b1[...], 0.0)
    x = jnp.maximum(_bdot(x, w2[...]) + b2[...], 0.0)
    m_ref_o[...] = jnp.maximum(_bdot(x, w3[...]) + b3[...], 0.0)


def _lstm_el(wih_t, whh_t, b, ep, agg, h, c):
    return pl.pallas_call(
        _lstm_el_body,
        grid=(_GRID,),
        in_specs=[_rows(_D), _rows(_D), _rows(_D),
                  _full((_D, 4 * _D)), _full((_D, 4 * _D)),
                  _full((1, 4 * _D)),
                  _full((_D, _D)), _full((1, _D)),
                  _full((_D, _D)), _full((1, _D)),
                  _full((_D, _D)), _full((1, _D))],
        out_specs=[_rows(_D), _rows(_D), _rows(_D)],
        out_shape=[jax.ShapeDtypeStruct((_N, _D), jnp.float32),
                   jax.ShapeDtypeStruct((_N, _D), jnp.float32),
                   jax.ShapeDtypeStruct((_N, _D), jnp.float32)],
    )(agg, h, c, wih_t, whh_t, b,
      ep["W1"], ep["b1"].reshape(1, _D), ep["W2"], ep["b2"].reshape(1, _D),
      ep["W3"], ep["b3"].reshape(1, _D))


def _lstm_c_body(x_ref, h_ref, c_ref, wih, whh, b, c_ref_o):
    gates = _bdot(x_ref[...], wih[...]) + _bdot(h_ref[...], whh[...]) + b[...]
    i = jax.nn.sigmoid(gates[:, 0:_D])
    f = jax.nn.sigmoid(gates[:, _D:2 * _D])
    g = jnp.tanh(gates[:, 2 * _D:3 * _D])
    c2 = f * c_ref[...] + i * g
    c_ref_o[...] = jnp.maximum(c2, 0.0)


def _lstm_c(wih_t, whh_t, b, agg, h, c):
    return pl.pallas_call(
        _lstm_c_body,
        grid=(_GRID,),
        in_specs=[_rows(_D), _rows(_D), _rows(_D),
                  _full((_D, 4 * _D)), _full((_D, 4 * _D)),
                  _full((1, 4 * _D))],
        out_specs=_rows(_D),
        out_shape=jax.ShapeDtypeStruct((_N, _D), jnp.float32),
    )(agg, h, c, wih_t, whh_t, b)


def _embed_el_body(xf_ref, emb_ref, w1, b1, w2, b2, w3, b3, cv_ref, m_ref):
    e0 = emb_ref[0:1, :]
    e1 = emb_ref[1:2, :]
    x = e0 + xf_ref[...] * (e1 - e0)
    cv_ref[...] = x
    x = jnp.maximum(_bdot(x, w1[...]) + b1[...], 0.0)
    x = jnp.maximum(_bdot(x, w2[...]) + b2[...], 0.0)
    m_ref[...] = jnp.maximum(_bdot(x, w3[...]) + b3[...], 0.0)


def _embed_el(xf, emb, ep):
    return pl.pallas_call(
        _embed_el_body,
        grid=(_GRID,),
        in_specs=[_rows(1), _full((2, _D)),
                  _full((_D, _D)), _full((1, _D)),
                  _full((_D, _D)), _full((1, _D)),
                  _full((_D, _D)), _full((1, _D))],
        out_specs=[_rows(_D), _rows(_D)],
        out_shape=[jax.ShapeDtypeStruct((_N, _D), jnp.float32),
                   jax.ShapeDtypeStruct((_N, _D), jnp.float32)],
    )(xf, emb, ep["W1"], ep["b1"].reshape(1, _D),
      ep["W2"], ep["b2"].reshape(1, _D), ep["W3"], ep["b3"].reshape(1, _D))


def _pool_body(feat_ref, gid_ref, pw_ref, out_ref, gmax_s, den_s, racc_s):
    ph = pl.program_id(0)
    i = pl.program_id(1)

    @pl.when((ph == 0) & (i == 0))
    def _():
        gmax_s[...] = jnp.full((1, _G), -1e30, jnp.float32)
        den_s[...] = jnp.zeros((_G, 1), jnp.float32)
        racc_s[...] = jnp.zeros((_G, _D), jnp.float32)

    feat = feat_ref[...]
    gid = gid_ref[...]
    gate = jnp.sum(feat * pw_ref[...], axis=1, keepdims=True)
    gidx = lax.broadcasted_iota(jnp.int32, (1, _G), 1).astype(jnp.float32)
    oh = gid == gidx

    @pl.when(ph == 0)
    def _():
        m = jnp.where(oh, gate, -1e30)
        gmax_s[...] = jnp.maximum(gmax_s[...], jnp.max(m, axis=0,
                                                       keepdims=True))

    @pl.when(ph == 1)
    def _():
        gsel = jnp.sum(jnp.where(oh, gmax_s[...], 0.0), axis=1, keepdims=True)
        eg = jnp.exp(gate - gsel)
        ohf = oh.astype(jnp.float32)
        den_s[...] += lax.dot_general(ohf, eg, (((0,), (0,)), ((), ())),
                                      preferred_element_type=jnp.float32)
        racc_s[...] += lax.dot_general(ohf, feat * eg,
                                       (((0,), (0,)), ((), ())),
                                       preferred_element_type=jnp.float32)

    @pl.when((ph == 1) & (i == _GRID - 1))
    def _():
        out_ref[...] = racc_s[...] / (den_s[...] + 1e-9)


def _pool(feat, gidf, pw_row):
    return pl.pallas_call(
        _pool_body,
        grid=(2, _GRID),
        in_specs=[pl.BlockSpec((_BR, _D), lambda p, i: (i, 0)),
                  pl.BlockSpec((_BR, 1), lambda p, i: (i, 0)),
                  pl.BlockSpec((1, _D), lambda p, i: (0, 0))],
        out_specs=pl.BlockSpec((_G, _D), lambda p, i: (0, 0)),
        out_shape=jax.ShapeDtypeStruct((_G, _D), jnp.float32),
        scratch_shapes=[pltpu.VMEM((1, _G), jnp.float32),
                        pltpu.VMEM((_G, 1), jnp.float32),
                        pltpu.VMEM((_G, _D), jnp.float32)],
    )(feat, gidf, pw_row)


def _mlp_body(r_ref, w1, b1, w2, b2, w3, b3, o_ref):
    x = jnp.maximum(jnp.dot(r_ref[...], w1[...],
                            preferred_element_type=jnp.float32) + b1[...], 0.0)
    x = jnp.maximum(jnp.dot(x, w2[...],
                            preferred_element_type=jnp.float32) + b2[...], 0.0)
    o_ref[...] = jnp.dot(x, w3[...],
                         preferred_element_type=jnp.float32) + b3[...]


def _mlp(p, readout):
    return pl.pallas_call(
        _mlp_body,
        in_specs=[pl.BlockSpec((_G, _D), lambda: (0, 0)),
                  pl.BlockSpec((_D, _D), lambda: (0, 0)),
                  pl.BlockSpec((1, _D), lambda: (0, 0)),
                  pl.BlockSpec((_D, _D), lambda: (0, 0)),
                  pl.BlockSpec((1, _D), lambda: (0, 0)),
                  pl.BlockSpec((_D, 2), lambda: (0, 0)),
                  pl.BlockSpec((1, 2), lambda: (0, 0))],
        out_specs=pl.BlockSpec((_G, 2), lambda: (0, 0)),
        out_shape=jax.ShapeDtypeStruct((_G, 2), jnp.float32),
    )(readout, p["mlp_W1"], p["mlp_b1"].reshape(1, _D),
      p["mlp_W2"], p["mlp_b2"].reshape(1, _D),
      p["mlp_W3"], p["mlp_b3"].reshape(1, 2))


def _pad_edges(src, dst):
    npad = _EPAD - _E
    src_p = jnp.concatenate(
        [src.astype(jnp.int32), (jnp.arange(npad, dtype=jnp.int32) * 97) % _N])
    dst_p = jnp.concatenate(
        [dst.astype(jnp.int32),
         _N + jnp.arange(npad, dtype=jnp.int32) % (_ACC_ROWS - _N)])
    return src_p, dst_p


def kernel(params, x_var, x_clause, v2c_src, v2c_dst, c2v_src, c2v_dst,
           clause_graph_id):
    p = params
    emb = p["embed"]

    cc = _embed(x_clause.astype(jnp.float32).reshape(_N, 1), emb)
    hc = cc

    v2c_s, v2c_d = _pad_edges(v2c_src, v2c_dst)
    c2v_s, c2v_d = _pad_edges(c2v_src, c2v_dst)
    zsrc = jnp.zeros((_ZROWS, 32), jnp.float32)

    l0 = (p["lstm0"]["Wih"].T, p["lstm0"]["Whh"].T,
          (p["lstm0"]["bih"] + p["lstm0"]["bhh"]).reshape(1, 4 * _D))
    l1 = (p["lstm1"]["Wih"].T, p["lstm1"]["Whh"].T,
          (p["lstm1"]["bih"] + p["lstm1"]["bhh"]).reshape(1, 4 * _D))

    cv, m = _embed_el(x_var.astype(jnp.float32).reshape(_N, 1), emb,
                      p["el_v2c"])
    hv = cv
    for step in range(_STEPS):
        agg = _segment_sum(m, v2c_s, v2c_d, zsrc)
        if step == _STEPS - 1:
            cc = _lstm_c(l0[0], l0[1], l0[2], agg, hc, cc)
            break
        hc, cc, m = _lstm_el(l0[0], l0[1], l0[2], p["el_c2v"], agg, hc, cc)
        agg = _segment_sum(m, c2v_s, c2v_d, zsrc)
        hv, cv, m = _lstm_el(l1[0], l1[1], l1[2], p["el_v2c"], agg, hv, cv)

    gidf = clause_graph_id.astype(jnp.float32).reshape(_N, 1)
    readout = _pool(cc, gidf, p["pool_W"].T)
    return _mlp(p, readout)


# speedup vs baseline: 1.0024x; 1.0024x over previous
import functools

import jax
import jax.numpy as jnp
from jax import lax
from jax.experimental import pallas as pl
from jax.experimental.pallas import tpu as pltpu
from jax.experimental.pallas import tpu_sc as plsc

_D = 128
_N = 50000
_E = 500000
_G = 64
_STEPS = 3

_BR = 1000
_GRID = _N // _BR

_EB = 128
_NTILES = 16
_GBLK = 2
_GPAIRS = 62
_GEDGES = _GBLK * _EB
_EPAD = _NTILES * 2 * _GPAIRS * _GEDGES
_ACC_ROWS = 50048
_ZROWS = _ACC_ROWS // _NTILES
_WOUT = 125
_OUT_ROWS = _N + 8


def _seg_body(m2, srcp, dstp, zsrc, out2, acc, sgA, dgA, sgB, dgB,
              idxb, dstb, rowsb, wbuf, widx, gsem, ssem, isem):
    cid = lax.axis_index("c")
    tid = lax.axis_index("s")
    iota = lax.iota(jnp.int32, 16)

    for p in range(2):
        k = cid * 2 + p
        pltpu.sync_copy(zsrc, acc.at[pl.ds(tid * _ZROWS, _ZROWS)])
        plsc.subcore_barrier()

        def stage(g, sg, dg, sem):
            base = (g * _NTILES + tid) * _GEDGES
            pltpu.async_copy(srcp.at[pl.ds(base, _GEDGES)], sg, sem)
            pltpu.async_copy(dstp.at[pl.ds(base, _GEDGES)], dg, sem)

        def wait_stage(sg, dg, sem):
            pltpu.make_async_copy(srcp.at[pl.ds(0, _GEDGES)], sg, sem).wait()
            pltpu.make_async_copy(dstp.at[pl.ds(0, _GEDGES)], dg, sem).wait()

        def drain_scatter(b):
            pltpu.make_async_copy(m2.at[pl.ds(0, _EB)], rowsb[b],
                                  ssem[b]).wait()

        def fire_set(i, sg, dg, b0):
            for b in (b0, b0 + 1):
                @pl.when(i > 0)
                def _():
                    drain_scatter(b)
                for j in range(_EB // 16):
                    o = (b - b0) * _EB + j * 16
                    s = sg[pl.ds(o, 16)]
                    idxb[b][pl.ds(j * 16, 16)] = s * 4 + k
                    dstb[b][pl.ds(j * 16, 16)] = dg[pl.ds(o, 16)]
                pltpu.async_copy(m2.at[idxb[b]], rowsb[b], gsem[b])

        def add_set(b0):
            for b in (b0, b0 + 1):
                pltpu.make_async_copy(m2.at[pl.ds(0, _EB)], rowsb[b],
                                      gsem[b]).wait()
                pltpu.async_copy(rowsb[b], acc.at[dstb[b]], ssem[b], add=True)

        stage(0, sgA, dgA, isem[0])
        stage(1, sgB, dgB, isem[1])

        def edge_pair(i, _):
            wait_stage(sgA, dgA, isem[0])
            fire_set(i, sgA, dgA, 0)
            wait_stage(sgB, dgB, isem[1])
            fire_set(i, sgB, dgB, 2)

            @pl.when(i < _GPAIRS - 1)
            def _():
                stage(2 * i + 2, sgA, dgA, isem[0])
                stage(2 * i + 3, sgB, dgB, isem[1])

            add_set(0)
            add_set(2)
            return 0

        lax.fori_loop(0, _GPAIRS, edge_pair, 0)
        for b in range(4):
            drain_scatter(b)
        plsc.subcore_barrier()

        def wdrain(buf, sem):
            pltpu.make_async_copy(m2.at[pl.ds(0, _EB)], buf, sem).wait()

        def write_block(i, buf, bidx, sem, first):
            row0 = tid * (25 * _WOUT) + i * _WOUT
            if first is None:
                @pl.when(i > 1)
                def _():
                    wdrain(buf, sem)
            elif not first:
                wdrain(buf, sem)
            pltpu.sync_copy(acc.at[pl.ds(row0, _EB)], buf)
            for j in range(_EB // 16):
                r = iota + (j * 16)
                v = (row0 + r) * 4 + k
                bidx[pl.ds(j * 16, 16)] = jnp.where(
                    r < _WOUT, v, _N * 4 + tid)
            pltpu.async_copy(buf, out2.at[bidx], sem)

        def write_pair(i, _):
            write_block(2 * i, wbuf, widx, ssem[0], None)
            write_block(2 * i + 1, rowsb[0], idxb[0], ssem[1], None)
            return 0

        lax.fori_loop(0, 12, write_pair, 0)
        write_block(24, wbuf, widx, ssem[0], False)
        wdrain(rowsb[0], ssem[1])
        wdrain(wbuf, ssem[0])
        plsc.subcore_barrier()


@functools.cache
def _seg_call():
    return functools.partial(
        pl.kernel,
        out_type=jax.ShapeDtypeStruct((_OUT_ROWS * 4, 32), jnp.float32),
        mesh=plsc.VectorSubcoreMesh(core_axis_name="c", subcore_axis_name="s"),
        compiler_params=pltpu.CompilerParams(use_tc_tiling_on_sc=False),
        scratch_types=[
            pltpu.VMEM_SHARED((_ACC_ROWS, 32), jnp.float32),
            pltpu.VMEM((_GEDGES,), jnp.int32),
            pltpu.VMEM((_GEDGES,), jnp.int32),
            pltpu.VMEM((_GEDGES,), jnp.int32),
            pltpu.VMEM((_GEDGES,), jnp.int32),
            [pltpu.VMEM((_EB,), jnp.int32) for _ in range(4)],
            [pltpu.VMEM((_EB,), jnp.int32) for _ in range(4)],
            [pltpu.VMEM((_EB, 32), jnp.float32) for _ in range(4)],
            pltpu.VMEM((_EB, 32), jnp.float32),
            pltpu.VMEM((_EB,), jnp.int32),
            [pltpu.SemaphoreType.DMA for _ in range(4)],
            [pltpu.SemaphoreType.DMA for _ in range(4)],
            [pltpu.SemaphoreType.DMA for _ in range(2)],
        ],
    )(_seg_body)


def _segment_sum(m, srcp, dstp, zsrc):
    m2 = m.reshape(_N * 4, 32)
    out2 = _seg_call()(m2, srcp, dstp, zsrc)
    return out2.reshape(_OUT_ROWS, _D)


def _full(shape):
    return pl.BlockSpec(shape, lambda i: (0, 0))


def _rows(width):
    return pl.BlockSpec((_BR, width), lambda i: (i, 0))


def _embed_body(xf_ref, emb_ref, o_ref):
    xf = xf_ref[...]
    e0 = emb_ref[0:1, :]
    e1 = emb_ref[1:2, :]
    o_ref[...] = e0 + xf * (e1 - e0)


def _embed(xf, emb):
    return pl.pallas_call(
        _embed_body,
        grid=(_GRID,),
        in_specs=[_rows(1), _full((2, _D))],
        out_specs=_rows(_D),
        out_shape=jax.ShapeDtypeStruct((_N, _D), jnp.float32),
    )(xf, emb)


def _bdot(x, w):
    return jnp.dot(x.astype(jnp.bfloat16), w.astype(jnp.bfloat16),
                   preferred_element_type=jnp.float32)


def _elayer_body(x_ref, w1, b1, w2, b2, w3, b3, o_ref):
    x = x_ref[...]
    x = jnp.maximum(_bdot(x, w1[...]) + b1[...], 0.0)
    x = jnp.maximum(_bdot(x, w2[...]) + b2[...], 0.0)
    x = jnp.maximum(_bdot(x, w3[...]) + b3[...], 0.0)
    o_ref[...] = x


def _elayer(ep, x):
    return pl.pallas_call(
        _elayer_body,
        grid=(_GRID,),
        in_specs=[_rows(_D), _full((_D, _D)), _full((1, _D)),
                  _full((_D, _D)), _full((1, _D)),
                  _full((_D, _D)), _full((1, _D))],
        out_specs=_rows(_D),
        out_shape=jax.ShapeDtypeStruct((_N, _D), jnp.float32),
    )(x, ep["W1"], ep["b1"].reshape(1, _D), ep["W2"], ep["b2"].reshape(1, _D),
      ep["W3"], ep["b3"].reshape(1, _D))


def _lstm_body(x_ref, h_ref, c_ref, wih, whh, b, h_ref_o, c_ref_o):
    gates = _bdot(x_ref[...], wih[...]) + _bdot(h_ref[...], whh[...]) + b[...]
    i = jax.nn.sigmoid(gates[:, 0:_D])
    f = jax.nn.sigmoid(gates[:, _D:2 * _D])
    g = jnp.tanh(gates[:, 2 * _D:3 * _D])
    o = jax.nn.sigmoid(gates[:, 3 * _D:4 * _D])
    c2 = f * c_ref[...] + i * g
    h_ref_o[...] = o * jnp.tanh(c2)
    c_ref_o[...] = jnp.maximum(c2, 0.0)


def _lstm_el_body(x_ref, h_ref, c_ref, wih, whh, b, w1, b1, w2, b2, w3, b3,
                  h_ref_o, c_ref_o, m_ref_o):
    gates = _bdot(x_ref[...], wih[...]) + _bdot(h_ref[...], whh[...]) + b[...]
    i = jax.nn.sigmoid(gates[:, 0:_D])
    f = jax.nn.sigmoid(gates[:, _D:2 * _D])
    g = jnp.tanh(gates[:, 2 * _D:3 * _D])
    o = jax.nn.sigmoid(gates[:, 3 * _D:4 * _D])
    c2 = f * c_ref[...] + i * g
    h_ref_o[...] = o * jnp.tanh(c2)
    x = jnp.maximum(c2, 0.0)
    c_ref_o[...] = x
    x = jnp.maximum(_bdot(x, w1[...]) + b1[...], 0.0)
    x = jnp.maximum(_bdot(x, w2[...]) + b2[...], 0.0)
    m_ref_o[...] = jnp.maximum(_bdot(x, w3[...]) + b3[...], 0.0)


def _lstm_el(wih_t, whh_t, b, ep, agg, h, c):
    return pl.pallas_call(
        _lstm_el_body,
        grid=(_GRID,),
        in_specs=[_rows(_D), _rows(_D), _rows(_D),
                  _full((_D, 4 * _D)), _full((_D, 4 * _D)),
                  _full((1, 4 * _D)),
                  _full((_D, _D)), _full((1, _D)),
                  _full((_D, _D)), _full((1, _D)),
                  _full((_D, _D)), _full((1, _D))],
        out_specs=[_rows(_D), _rows(_D), _rows(_D)],
        out_shape=[jax.ShapeDtypeStruct((_N, _D), jnp.float32),
                   jax.ShapeDtypeStruct((_N, _D), jnp.float32),
                   jax.ShapeDtypeStruct((_N, _D), jnp.float32)],
    )(agg, h, c, wih_t, whh_t, b,
      ep["W1"], ep["b1"].reshape(1, _D), ep["W2"], ep["b2"].reshape(1, _D),
      ep["W3"], ep["b3"].reshape(1, _D))


def _lstm_c_body(x_ref, h_ref, c_ref, wih, whh, b, c_ref_o):
    gates = _bdot(x_ref[...], wih[...]) + _bdot(h_ref[...], whh[...]) + b[...]
    i = jax.nn.sigmoid(gates[:, 0:_D])
    f = jax.nn.sigmoid(gates[:, _D:2 * _D])
    g = jnp.tanh(gates[:, 2 * _D:3 * _D])
    c2 = f * c_ref[...] + i * g
    c_ref_o[...] = jnp.maximum(c2, 0.0)


def _lstm_c(wih_t, whh_t, b, agg, h, c):
    return pl.pallas_call(
        _lstm_c_body,
        grid=(_GRID,),
        in_specs=[_rows(_D), _rows(_D), _rows(_D),
                  _full((_D, 4 * _D)), _full((_D, 4 * _D)),
                  _full((1, 4 * _D))],
        out_specs=_rows(_D),
        out_shape=jax.ShapeDtypeStruct((_N, _D), jnp.float32),
    )(agg, h, c, wih_t, whh_t, b)


def _embed_el_body(xf_ref, emb_ref, w1, b1, w2, b2, w3, b3, cv_ref, m_ref):
    e0 = emb_ref[0:1, :]
    e1 = emb_ref[1:2, :]
    x = e0 + xf_ref[...] * (e1 - e0)
    cv_ref[...] = x
    x = jnp.maximum(_bdot(x, w1[...]) + b1[...], 0.0)
    x = jnp.maximum(_bdot(x, w2[...]) + b2[...], 0.0)
    m_ref[...] = jnp.maximum(_bdot(x, w3[...]) + b3[...], 0.0)


def _embed_el(xf, emb, ep):
    return pl.pallas_call(
        _embed_el_body,
        grid=(_GRID,),
        in_specs=[_rows(1), _full((2, _D)),
                  _full((_D, _D)), _full((1, _D)),
                  _full((_D, _D)), _full((1, _D)),
                  _full((_D, _D)), _full((1, _D))],
        out_specs=[_rows(_D), _rows(_D)],
        out_shape=[jax.ShapeDtypeStruct((_N, _D), jnp.float32),
                   jax.ShapeDtypeStruct((_N, _D), jnp.float32)],
    )(xf, emb, ep["W1"], ep["b1"].reshape(1, _D),
      ep["W2"], ep["b2"].reshape(1, _D), ep["W3"], ep["b3"].reshape(1, _D))


def _pool_body(feat_ref, gid_ref, pw_ref, out_ref, gmax_s, den_s, racc_s):
    ph = pl.program_id(0)
    i = pl.program_id(1)

    @pl.when((ph == 0) & (i == 0))
    def _():
        gmax_s[...] = jnp.full((1, _G), -1e30, jnp.float32)
        den_s[...] = jnp.zeros((_G, 1), jnp.float32)
        racc_s[...] = jnp.zeros((_G, _D), jnp.float32)

    feat = feat_ref[...]
    gid = gid_ref[...]
    gate = jnp.sum(feat * pw_ref[...], axis=1, keepdims=True)
    gidx = lax.broadcasted_iota(jnp.int32, (1, _G), 1).astype(jnp.float32)
    oh = gid == gidx

    @pl.when(ph == 0)
    def _():
        m = jnp.where(oh, gate, -1e30)
        gmax_s[...] = jnp.maximum(gmax_s[...], jnp.max(m, axis=0,
                                                       keepdims=True))

    @pl.when(ph == 1)
    def _():
        gsel = jnp.sum(jnp.where(oh, gmax_s[...], 0.0), axis=1, keepdims=True)
        eg = jnp.exp(gate - gsel)
        ohf = oh.astype(jnp.float32)
        den_s[...] += lax.dot_general(ohf, eg, (((0,), (0,)), ((), ())),
                                      preferred_element_type=jnp.float32)
        racc_s[...] += lax.dot_general(ohf, feat * eg,
                                       (((0,), (0,)), ((), ())),
                                       preferred_element_type=jnp.float32)

    @pl.when((ph == 1) & (i == _GRID - 1))
    def _():
        out_ref[...] = racc_s[...] / (den_s[...] + 1e-9)


def _pool(feat, gidf, pw_row):
    return pl.pallas_call(
        _pool_body,
        grid=(2, _GRID),
        in_specs=[pl.BlockSpec((_BR, _D), lambda p, i: (i, 0)),
                  pl.BlockSpec((_BR, 1), lambda p, i: (i, 0)),
                  pl.BlockSpec((1, _D), lambda p, i: (0, 0))],
        out_specs=pl.BlockSpec((_G, _D), lambda p, i: (0, 0)),
        out_shape=jax.ShapeDtypeStruct((_G, _D), jnp.float32),
        scratch_shapes=[pltpu.VMEM((1, _G), jnp.float32),
                        pltpu.VMEM((_G, 1), jnp.float32),
                        pltpu.VMEM((_G, _D), jnp.float32)],
    )(feat, gidf, pw_row)


def _mlp_body(r_ref, w1, b1, w2, b2, w3, b3, o_ref):
    x = jnp.maximum(jnp.dot(r_ref[...], w1[...],
                            preferred_element_type=jnp.float32) + b1[...], 0.0)
    x = jnp.maximum(jnp.dot(x, w2[...],
                            preferred_element_type=jnp.float32) + b2[...], 0.0)
    o_ref[...] = jnp.dot(x, w3[...],
                         preferred_element_type=jnp.float32) + b3[...]


def _mlp(p, readout):
    return pl.pallas_call(
        _mlp_body,
        in_specs=[pl.BlockSpec((_G, _D), lambda: (0, 0)),
                  pl.BlockSpec((_D, _D), lambda: (0, 0)),
                  pl.BlockSpec((1, _D), lambda: (0, 0)),
                  pl.BlockSpec((_D, _D), lambda: (0, 0)),
                  pl.BlockSpec((1, _D), lambda: (0, 0)),
                  pl.BlockSpec((_D, 2), lambda: (0, 0)),
                  pl.BlockSpec((1, 2), lambda: (0, 0))],
        out_specs=pl.BlockSpec((_G, 2), lambda: (0, 0)),
        out_shape=jax.ShapeDtypeStruct((_G, 2), jnp.float32),
    )(readout, p["mlp_W1"], p["mlp_b1"].reshape(1, _D),
      p["mlp_W2"], p["mlp_b2"].reshape(1, _D),
      p["mlp_W3"], p["mlp_b3"].reshape(1, 2))


def _pad_edges(src, dst):
    npad = _EPAD - _E
    src_p = jnp.concatenate(
        [src.astype(jnp.int32), (jnp.arange(npad, dtype=jnp.int32) * 97) % _N])
    dst_p = jnp.concatenate(
        [dst.astype(jnp.int32),
         _N + jnp.arange(npad, dtype=jnp.int32) % (_ACC_ROWS - _N)])
    return src_p, dst_p


def kernel(params, x_var, x_clause, v2c_src, v2c_dst, c2v_src, c2v_dst,
           clause_graph_id):
    p = params
    emb = p["embed"]

    cc = _embed(x_clause.astype(jnp.float32).reshape(_N, 1), emb)
    hc = cc

    v2c_s, v2c_d = _pad_edges(v2c_src, v2c_dst)
    c2v_s, c2v_d = _pad_edges(c2v_src, c2v_dst)
    zsrc = jnp.zeros((_ZROWS, 32), jnp.float32)

    l0 = (p["lstm0"]["Wih"].T, p["lstm0"]["Whh"].T,
          (p["lstm0"]["bih"] + p["lstm0"]["bhh"]).reshape(1, 4 * _D))
    l1 = (p["lstm1"]["Wih"].T, p["lstm1"]["Whh"].T,
          (p["lstm1"]["bih"] + p["lstm1"]["bhh"]).reshape(1, 4 * _D))

    cv, m = _embed_el(x_var.astype(jnp.float32).reshape(_N, 1), emb,
                      p["el_v2c"])
    hv = cv
    for step in range(_STEPS):
        agg = _segment_sum(m, v2c_s, v2c_d, zsrc)
        if step == _STEPS - 1:
            cc = _lstm_c(l0[0], l0[1], l0[2], agg, hc, cc)
            break
        hc, cc, m = _lstm_el(l0[0], l0[1], l0[2], p["el_c2v"], agg, hc, cc)
        agg = _segment_sum(m, c2v_s, c2v_d, zsrc)
        hv, cv, m = _lstm_el(l1[0], l1[1], l1[2], p["el_v2c"], agg, hv, cv)

    gidf = clause_graph_id.astype(jnp.float32).reshape(_N, 1)
    readout = _pool(cc, gidf, p["pool_W"].T)
    return _mlp(p, readout)
